# baseline (speedup 1.0000x reference)
import functools

import jax
import jax.numpy as jnp
from jax import lax
from jax.experimental import pallas as pl
from jax.experimental.pallas import tpu as pltpu
from jax.experimental.pallas import tpu_sc as plsc

N_NODES = 10000
N_EDGES = 320000
D_IN = 128
D_H = 64
N_STEPS = 3
N_BLOCKS = 3
N_OUT = 1

NW = 32
EPW = N_EDGES // NW
CH = 80
NCH = EPW // CH

_MESH = plsc.VectorSubcoreMesh(core_axis_name="c", subcore_axis_name="s",
                               num_cores=2, num_subcores=16)

f32 = jnp.float32
bf16 = jnp.bfloat16


def _gate(z):
    v, g = jnp.split(z, 2, axis=-1)
    return v * jax.nn.sigmoid(v) * jax.nn.sigmoid(g)


@functools.partial(
    pl.kernel,
    out_type=(
        jax.ShapeDtypeStruct((N_EDGES, D_H), f32),
        jax.ShapeDtypeStruct((N_EDGES, D_H), f32),
    ),
    mesh=_MESH,
    compiler_params=pltpu.CompilerParams(use_tc_tiling_on_sc=False),
    scratch_types=[
        pltpu.VMEM((NCH, CH), jnp.int32),
        pltpu.VMEM((NCH, CH), jnp.int32),
        pltpu.VMEM((2, CH, D_H), f32),
        pltpu.VMEM((2, CH, D_H), f32),
        pltpu.SemaphoreType.DMA((2,)),
        pltpu.SemaphoreType.DMA((2,)),
        pltpu.SemaphoreType.DMA((2,)),
        pltpu.SemaphoreType.DMA((2,)),
    ],
)
def _sc_gather(a_hbm, b_hbm, src_hbm, dst_hbm, gs_hbm, gd_hbm,
               src_v, dst_v, bufs, bufd, gsem_s, gsem_d, wsem_s, wsem_d):
    wid = lax.axis_index("s") * 2 + lax.axis_index("c")
    pltpu.sync_copy(src_hbm.at[wid], src_v)
    pltpu.sync_copy(dst_hbm.at[wid], dst_v)

    def start_gather(j, p):
        pltpu.async_copy(a_hbm.at[src_v.at[j]], bufs.at[p], gsem_s.at[p])
        pltpu.async_copy(b_hbm.at[dst_v.at[j]], bufd.at[p], gsem_d.at[p])

    def wait_gather(p):
        pltpu.make_async_copy(a_hbm.at[src_v.at[0]], bufs.at[p],
                              gsem_s.at[p]).wait()
        pltpu.make_async_copy(b_hbm.at[dst_v.at[0]], bufd.at[p],
                              gsem_d.at[p]).wait()

    def start_write(j, p):
        base = wid * EPW + j * CH
        pltpu.async_copy(bufs.at[p], gs_hbm.at[pl.ds(base, CH)], wsem_s.at[p])
        pltpu.async_copy(bufd.at[p], gd_hbm.at[pl.ds(base, CH)], wsem_d.at[p])

    def wait_write(p):
        pltpu.make_async_copy(bufs.at[p], gs_hbm.at[pl.ds(0, CH)],
                              wsem_s.at[p]).wait()
        pltpu.make_async_copy(bufd.at[p], gd_hbm.at[pl.ds(0, CH)],
                              wsem_d.at[p]).wait()

    start_gather(0, 0)

    def body(j, _):
        p = lax.rem(j, 2)
        q = 1 - p
        wait_gather(p)

        @pl.when(j + 1 < NCH)
        def _():
            @pl.when(j >= 1)
            def _():
                wait_write(q)

            start_gather(j + 1, q)

        start_write(j, p)
        return 0

    lax.fori_loop(0, NCH, body, 0)
    wait_write((NCH - 1) % 2)
    wait_write(NCH % 2)


@functools.partial(
    pl.kernel,
    out_type=jax.ShapeDtypeStruct((2, N_NODES, D_H), f32),
    mesh=_MESH,
    scratch_types=[
        pltpu.VMEM((NCH, CH), jnp.int32),
        pltpu.VMEM((2, CH, D_H), f32),
        pltpu.VMEM_SHARED((N_NODES, D_H), f32),
        pltpu.SemaphoreType.DMA((2,)),
        pltpu.SemaphoreType.DMA((2,)),
    ],
)
def _sc_scatter(m_hbm, dst_hbm, zero_hbm, out_hbm, dst_v, buf, acc,
                rsem, ssem):
    cid = lax.axis_index("c")
    sid = lax.axis_index("s")
    wid = sid * 2 + cid

    @pl.when(sid == 0)
    def _():
        pltpu.sync_copy(zero_hbm, acc)

    pltpu.sync_copy(dst_hbm.at[wid], dst_v)
    plsc.subcore_barrier()

    def start_read(j, p):
        base = wid * EPW + j * CH
        pltpu.async_copy(m_hbm.at[pl.ds(base, CH)], buf.at[p], rsem.at[p])

    def wait_read(p):
        pltpu.make_async_copy(m_hbm.at[pl.ds(0, CH)], buf.at[p],
                              rsem.at[p]).wait()

    def wait_scatter(p):
        pltpu.make_async_copy(buf.at[p], acc.at[dst_v.at[0]],
                              ssem.at[p]).wait()

    start_read(0, 0)

    def body(j, _):
        p = lax.rem(j, 2)
        q = 1 - p
        wait_read(p)

        @pl.when(j + 1 < NCH)
        def _():
            @pl.when(j >= 1)
            def _():
                wait_scatter(q)

            start_read(j + 1, q)

        pltpu.async_copy(buf.at[p], acc.at[dst_v.at[j]], ssem.at[p],
                         add=True)
        return 0

    lax.fori_loop(0, NCH, body, 0)
    wait_scatter((NCH - 1) % 2)
    wait_scatter(NCH % 2)
    plsc.subcore_barrier()

    @pl.when(sid == 0)
    def _():
        pltpu.sync_copy(acc, out_hbm.at[cid])


def _ab(h, w1, b1):
    a = jnp.dot(h, w1[:D_H], preferred_element_type=f32) + b1
    b = jnp.dot(h, w1[D_H:], preferred_element_type=f32)
    return a.astype(bf16), b.astype(bf16)


def _embed_body(x_ref, w_ref, b_ref, w1_ref, b1_ref, h_ref, a_ref, bb_ref):
    h = jnp.dot(x_ref[...], w_ref[...], preferred_element_type=f32) + b_ref[...]
    h_ref[...] = h
    a_ref[...], bb_ref[...] = _ab(h, w1_ref[...], b1_ref[...])


def _embed_pre(x, Wemb, bemb, W1, b1):
    return pl.pallas_call(
        _embed_body,
        out_shape=(
            jax.ShapeDtypeStruct((N_NODES, D_H), f32),
            jax.ShapeDtypeStruct((N_NODES, 2 * D_H), bf16),
            jax.ShapeDtypeStruct((N_NODES, 2 * D_H), bf16),
        ),
    )(x, Wemb, bemb.reshape(1, D_H), W1, b1.reshape(1, 2 * D_H))


BE = 3200


def _edge_body(gs_ref, gd_ref, w2_ref, b2_ref, w3_ref, b3_ref, o_ref):
    m1 = _gate(gs_ref[...].astype(f32) + gd_ref[...].astype(f32))
    m2 = _gate(jnp.dot(m1, w2_ref[...], preferred_element_type=f32)
               + b2_ref[...])
    o_ref[...] = (jnp.dot(m2, w3_ref[...], preferred_element_type=f32)
                  + b3_ref[...])


def _edge_mlp(gs, gd, W2, b2, W3, b3):
    nblk = N_EDGES // BE
    eb = pl.BlockSpec((BE, 2 * D_H), lambda i: (i, 0))
    full = lambda *shape: pl.BlockSpec(shape, lambda i: (0,) * len(shape))
    return pl.pallas_call(
        _edge_body,
        grid=(nblk,),
        in_specs=[eb, eb,
                  full(D_H, 2 * D_H), full(1, 2 * D_H),
                  full(D_H, D_H), full(1, D_H)],
        out_specs=pl.BlockSpec((BE, D_H), lambda i: (i, 0)),
        out_shape=jax.ShapeDtypeStruct((N_EDGES, D_H), f32),
    )(gs, gd, W2, b2.reshape(1, -1), W3, b3.reshape(1, -1))


def _node_body(s_ref, h_ref, wn_ref, bn_ref, w1_ref, b1_ref,
               o_ref, a_ref, bb_ref):
    scale = 1.0 / (N_EDGES - 1)
    m_i = (s_ref[0] + s_ref[1]) * scale
    h = h_ref[...]
    wn = wn_ref[...]
    h_new = (jnp.dot(m_i, wn[:D_H], preferred_element_type=f32)
             + jnp.dot(h, wn[D_H:], preferred_element_type=f32)
             + bn_ref[...] + h)
    o_ref[...] = h_new
    a_ref[...], bb_ref[...] = _ab(h_new, w1_ref[...], b1_ref[...])


def _node_update_pre(s_parts, h, Wn, bn, W1, b1):
    return pl.pallas_call(
        _node_body,
        out_shape=(
            jax.ShapeDtypeStruct((N_NODES, D_H), f32),
            jax.ShapeDtypeStruct((N_NODES, 2 * D_H), bf16),
            jax.ShapeDtypeStruct((N_NODES, 2 * D_H), bf16),
        ),
    )(s_parts, h, Wn, bn.reshape(1, D_H), W1, b1.reshape(1, 2 * D_H))


def _final_body(s_ref, h_ref, wn_ref, bn_ref, wd_ref, bd_ref,
                wdf_ref, bdf_ref, o_ref):
    scale = 1.0 / (N_EDGES - 1)
    m_i = (s_ref[0] + s_ref[1]) * scale
    h = h_ref[...]
    wn = wn_ref[...]
    h = (jnp.dot(m_i, wn[:D_H], preferred_element_type=f32)
         + jnp.dot(h, wn[D_H:], preferred_element_type=f32)
         + bn_ref[...] + h)
    for b in range(N_BLOCKS):
        h = _gate(jnp.dot(h, wd_ref[b], preferred_element_type=f32)
                  + bd_ref[b])
    o_ref[...] = (jnp.sum(h * wdf_ref[...], axis=1, keepdims=True)
                  + bdf_ref[...])


def _final_decode(s_parts, h, Wn, bn, Wd, bd, Wdf, bdf):
    return pl.pallas_call(
        _final_body,
        out_shape=jax.ShapeDtypeStruct((N_NODES, N_OUT), f32),
    )(s_parts, h, Wn, bn.reshape(1, D_H), Wd,
      bd.reshape(N_BLOCKS, 1, 2 * D_H), Wdf.reshape(1, D_H),
      bdf.reshape(1, N_OUT))


def kernel(x, edge_index, Wemb, bemb, We1, be1, We2, be2, We3, be3,
           Wn, bn, Wd, bd, Wdf, bdf):
    src = edge_index[0].astype(jnp.int32).reshape(NW, NCH, CH)
    dst = edge_index[1].astype(jnp.int32).reshape(NW, NCH, CH)
    zeros = jnp.zeros((N_NODES, D_H), dtype=f32)

    def pack(t):
        return jax.lax.bitcast_convert_type(t.reshape(N_NODES, D_H, 2), f32)

    def unpack(t):
        return jax.lax.bitcast_convert_type(t, bf16).reshape(N_EDGES, 2 * D_H)

    h, a, b = _embed_pre(x, Wemb, bemb, We1[0], be1[0])
    for s in range(N_STEPS):
        gs, gd = _sc_gather(pack(a), pack(b), src, dst)
        gs, gd = unpack(gs), unpack(gd)
        m3 = _edge_mlp(gs, gd, We2[s], be2[s], We3[s], be3[s])
        s_parts = _sc_scatter(m3, dst, zeros)
        if s + 1 < N_STEPS:
            h, a, b = _node_update_pre(s_parts, h, Wn[s], bn[s],
                                       We1[s + 1], be1[s + 1])
        else:
            return _final_decode(s_parts, h, Wn[s], bn[s], Wd, bd, Wdf, bdf)


# speedup vs baseline: 3.0396x; 3.0396x over previous
import functools

import jax
import jax.numpy as jnp
from jax import lax
from jax.experimental import pallas as pl
from jax.experimental.pallas import tpu as pltpu
from jax.experimental.pallas import tpu_sc as plsc

N_NODES = 10000
N_EDGES = 320000
D_IN = 128
D_H = 64
N_STEPS = 3
N_BLOCKS = 3
N_OUT = 1

NW = 32

_MESH = plsc.VectorSubcoreMesh(core_axis_name="c", subcore_axis_name="s",
                               num_cores=2, num_subcores=16)

f32 = jnp.float32


def _gate(z):
    v, g = jnp.split(z, 2, axis=-1)
    return v * jax.nn.sigmoid(v) * jax.nn.sigmoid(g)


K = 2
EC = N_EDGES // K
EPW = EC // NW
CH = 40
NCH = EPW // CH


def _make_gather():
    @functools.partial(
        pl.kernel,
        out_type=(
            jax.ShapeDtypeStruct((EC, 2 * D_H), f32),
            jax.ShapeDtypeStruct((EC, 2 * D_H), f32),
        ),
        mesh=_MESH,
        scratch_types=[
            pltpu.VMEM((NCH, CH), jnp.int32),
            pltpu.VMEM((NCH, CH), jnp.int32),
            pltpu.VMEM((2, CH, 2 * D_H), f32),
            pltpu.VMEM((2, CH, 2 * D_H), f32),
            pltpu.SemaphoreType.DMA((2,)),
            pltpu.SemaphoreType.DMA((2,)),
            pltpu.SemaphoreType.DMA((2,)),
            pltpu.SemaphoreType.DMA((2,)),
        ],
    )
    def _sc_gather(a_hbm, b_hbm, src_hbm, dst_hbm, gs_hbm, gd_hbm,
                   src_v, dst_v, bufs, bufd, gsem_s, gsem_d, wsem_s, wsem_d):
        wid = lax.axis_index("s") * 2 + lax.axis_index("c")
        pltpu.sync_copy(src_hbm.at[wid], src_v)
        pltpu.sync_copy(dst_hbm.at[wid], dst_v)

        def start_gather(j, p):
            pltpu.async_copy(a_hbm.at[src_v.at[j]], bufs.at[p], gsem_s.at[p])
            pltpu.async_copy(b_hbm.at[dst_v.at[j]], bufd.at[p], gsem_d.at[p])

        def wait_gather(p):
            pltpu.make_async_copy(a_hbm.at[src_v.at[0]], bufs.at[p],
                                  gsem_s.at[p]).wait()
            pltpu.make_async_copy(b_hbm.at[dst_v.at[0]], bufd.at[p],
                                  gsem_d.at[p]).wait()

        def start_write(j, p):
            base = wid * EPW + j * CH
            pltpu.async_copy(bufs.at[p], gs_hbm.at[pl.ds(base, CH)],
                             wsem_s.at[p])
            pltpu.async_copy(bufd.at[p], gd_hbm.at[pl.ds(base, CH)],
                             wsem_d.at[p])

        def wait_write(p):
            pltpu.make_async_copy(bufs.at[p], gs_hbm.at[pl.ds(0, CH)],
                                  wsem_s.at[p]).wait()
            pltpu.make_async_copy(bufd.at[p], gd_hbm.at[pl.ds(0, CH)],
                                  wsem_d.at[p]).wait()

        start_gather(0, 0)

        def body(j, _):
            p = lax.rem(j, 2)
            q = 1 - p
            wait_gather(p)

            @pl.when(j + 1 < NCH)
            def _():
                @pl.when(j >= 1)
                def _():
                    wait_write(q)

                start_gather(j + 1, q)

            start_write(j, p)
            return 0

        lax.fori_loop(0, NCH, body, 0)
        wait_write((NCH - 1) % 2)
        wait_write(NCH % 2)

    return _sc_gather


def _make_scatter():
    @functools.partial(
        pl.kernel,
        out_type=jax.ShapeDtypeStruct((2, N_NODES, D_H), f32),
        mesh=_MESH,
        scratch_types=[
            pltpu.VMEM((NCH, CH), jnp.int32),
            pltpu.VMEM((2, CH, D_H), f32),
            pltpu.VMEM_SHARED((N_NODES, D_H), f32),
            pltpu.SemaphoreType.DMA((2,)),
            pltpu.SemaphoreType.DMA((2,)),
        ],
    )
    def _sc_scatter(m_hbm, dst_hbm, zero_hbm, out_hbm, dst_v, buf, acc,
                    rsem, ssem):
        cid = lax.axis_index("c")
        sid = lax.axis_index("s")
        wid = sid * 2 + cid

        @pl.when(sid == 0)
        def _():
            pltpu.sync_copy(zero_hbm, acc)

        pltpu.sync_copy(dst_hbm.at[wid], dst_v)
        plsc.subcore_barrier()

        def start_read(j, p):
            base = wid * EPW + j * CH
            pltpu.async_copy(m_hbm.at[pl.ds(base, CH)], buf.at[p], rsem.at[p])

        def wait_read(p):
            pltpu.make_async_copy(m_hbm.at[pl.ds(0, CH)], buf.at[p],
                                  rsem.at[p]).wait()

        def wait_scatter(p):
            pltpu.make_async_copy(buf.at[p], acc.at[dst_v.at[0]],
                                  ssem.at[p]).wait()

        start_read(0, 0)

        def body(j, _):
            p = lax.rem(j, 2)
            q = 1 - p
            wait_read(p)

            @pl.when(j + 1 < NCH)
            def _():
                @pl.when(j >= 1)
                def _():
                    wait_scatter(q)

                start_read(j + 1, q)

            pltpu.async_copy(buf.at[p], acc.at[dst_v.at[j]], ssem.at[p],
                             add=True)
            return 0

        lax.fori_loop(0, NCH, body, 0)
        wait_scatter((NCH - 1) % 2)
        wait_scatter(NCH % 2)
        plsc.subcore_barrier()

        @pl.when(sid == 0)
        def _():
            pltpu.sync_copy(acc, out_hbm.at[cid])

    return _sc_scatter


_sc_gather = _make_gather()
_sc_scatter = _make_scatter()


def _ab(h, w1, b1):
    a = jnp.dot(h, w1[:D_H], preferred_element_type=f32) + b1
    b = jnp.dot(h, w1[D_H:], preferred_element_type=f32)
    return a, b


def _embed_body(x_ref, w_ref, b_ref, w1_ref, b1_ref, h_ref, a_ref, bb_ref):
    h = jnp.dot(x_ref[...], w_ref[...], preferred_element_type=f32) + b_ref[...]
    h_ref[...] = h
    a_ref[...], bb_ref[...] = _ab(h, w1_ref[...], b1_ref[...])


def _embed_pre(x, Wemb, bemb, W1, b1):
    return pl.pallas_call(
        _embed_body,
        out_shape=(
            jax.ShapeDtypeStruct((N_NODES, D_H), f32),
            jax.ShapeDtypeStruct((N_NODES, 2 * D_H), f32),
            jax.ShapeDtypeStruct((N_NODES, 2 * D_H), f32),
        ),
    )(x, Wemb, bemb.reshape(1, D_H), W1, b1.reshape(1, 2 * D_H))


BE = 3200


def _edge_body(gs_ref, gd_ref, w2_ref, b2_ref, w3_ref, b3_ref, o_ref):
    m1 = _gate(gs_ref[...] + gd_ref[...])
    m2 = _gate(jnp.dot(m1, w2_ref[...], preferred_element_type=f32)
               + b2_ref[...])
    o_ref[...] = (jnp.dot(m2, w3_ref[...], preferred_element_type=f32)
                  + b3_ref[...])


def _edge_mlp(gs, gd, W2, b2, W3, b3):
    nblk = EC // BE
    eb = pl.BlockSpec((BE, 2 * D_H), lambda i: (i, 0))
    full = lambda *shape: pl.BlockSpec(shape, lambda i: (0,) * len(shape))
    return pl.pallas_call(
        _edge_body,
        grid=(nblk,),
        in_specs=[eb, eb,
                  full(D_H, 2 * D_H), full(1, 2 * D_H),
                  full(D_H, D_H), full(1, D_H)],
        out_specs=pl.BlockSpec((BE, D_H), lambda i: (i, 0)),
        out_shape=jax.ShapeDtypeStruct((EC, D_H), f32),
    )(gs, gd, W2, b2.reshape(1, -1), W3, b3.reshape(1, -1))


def _node_body(s_ref, h_ref, wn_ref, bn_ref, w1_ref, b1_ref,
               o_ref, a_ref, bb_ref):
    scale = 1.0 / (N_EDGES - 1)
    m_i = sum(s_ref[i] for i in range(2 * K)) * scale
    h = h_ref[...]
    wn = wn_ref[...]
    h_new = (jnp.dot(m_i, wn[:D_H], preferred_element_type=f32)
             + jnp.dot(h, wn[D_H:], preferred_element_type=f32)
             + bn_ref[...] + h)
    o_ref[...] = h_new
    a_ref[...], bb_ref[...] = _ab(h_new, w1_ref[...], b1_ref[...])


def _node_update_pre(s_parts, h, Wn, bn, W1, b1):
    return pl.pallas_call(
        _node_body,
        out_shape=(
            jax.ShapeDtypeStruct((N_NODES, D_H), f32),
            jax.ShapeDtypeStruct((N_NODES, 2 * D_H), f32),
            jax.ShapeDtypeStruct((N_NODES, 2 * D_H), f32),
        ),
    )(s_parts, h, Wn, bn.reshape(1, D_H), W1, b1.reshape(1, 2 * D_H))


def _final_body(s_ref, h_ref, wn_ref, bn_ref, wd_ref, bd_ref,
                wdf_ref, bdf_ref, o_ref):
    scale = 1.0 / (N_EDGES - 1)
    m_i = sum(s_ref[i] for i in range(2 * K)) * scale
    h = h_ref[...]
    wn = wn_ref[...]
    h = (jnp.dot(m_i, wn[:D_H], preferred_element_type=f32)
         + jnp.dot(h, wn[D_H:], preferred_element_type=f32)
         + bn_ref[...] + h)
    for b in range(N_BLOCKS):
        h = _gate(jnp.dot(h, wd_ref[b], preferred_element_type=f32)
                  + bd_ref[b])
    o_ref[...] = (jnp.sum(h * wdf_ref[...], axis=1, keepdims=True)
                  + bdf_ref[...])


def _final_decode(s_parts, h, Wn, bn, Wd, bd, Wdf, bdf):
    return pl.pallas_call(
        _final_body,
        out_shape=jax.ShapeDtypeStruct((N_NODES, N_OUT), f32),
    )(s_parts, h, Wn, bn.reshape(1, D_H), Wd,
      bd.reshape(N_BLOCKS, 1, 2 * D_H), Wdf.reshape(1, D_H),
      bdf.reshape(1, N_OUT))


def kernel(x, edge_index, Wemb, bemb, We1, be1, We2, be2, We3, be3,
           Wn, bn, Wd, bd, Wdf, bdf):
    src = edge_index[0].astype(jnp.int32).reshape(K, NW, NCH, CH)
    dst = edge_index[1].astype(jnp.int32).reshape(K, NW, NCH, CH)
    zeros = jnp.zeros((N_NODES, D_H), dtype=f32)

    h, a, b = _embed_pre(x, Wemb, bemb, We1[0], be1[0])
    for s in range(N_STEPS):
        parts = []
        for k in range(K):
            gs, gd = _sc_gather(a, b, src[k], dst[k])
            m3 = _edge_mlp(gs, gd, We2[s], be2[s], We3[s], be3[s])
            parts.append(_sc_scatter(m3, dst[k], zeros))
        s_parts = jnp.concatenate(parts, axis=0)
        if s + 1 < N_STEPS:
            h, a, b = _node_update_pre(s_parts, h, Wn[s], bn[s],
                                       We1[s + 1], be1[s + 1])
        else:
            return _final_decode(s_parts, h, Wn[s], bn[s], Wd, bd, Wdf, bdf)


# speedup vs baseline: 3.8822x; 1.2772x over previous
import functools

import jax
import jax.numpy as jnp
from jax import lax
from jax.experimental import pallas as pl
from jax.experimental.pallas import tpu as pltpu
from jax.experimental.pallas import tpu_sc as plsc

N_NODES = 10000
N_EDGES = 320000
D_IN = 128
D_H = 64
N_STEPS = 3
N_BLOCKS = 3
N_OUT = 1

NW = 32
EPW = N_EDGES // NW
CH = 80
NCH = EPW // CH

_MESH = plsc.VectorSubcoreMesh(core_axis_name="c", subcore_axis_name="s",
                               num_cores=2, num_subcores=16)

f32 = jnp.float32


def _gate(z):
    v, g = jnp.split(z, 2, axis=-1)
    return v * jax.nn.sigmoid(v) * jax.nn.sigmoid(g)


@functools.partial(
    pl.kernel,
    out_type=(
        jax.ShapeDtypeStruct((N_EDGES, 2 * D_H), f32),
        jax.ShapeDtypeStruct((N_EDGES, 2 * D_H), f32),
    ),
    mesh=_MESH,
    scratch_types=[
        pltpu.VMEM((NCH, CH), jnp.int32),
        pltpu.VMEM((NCH, CH), jnp.int32),
        pltpu.VMEM((3, CH, 2 * D_H), f32),
        pltpu.VMEM((3, CH, 2 * D_H), f32),
        pltpu.SemaphoreType.DMA((3,)),
        pltpu.SemaphoreType.DMA((3,)),
        pltpu.SemaphoreType.DMA((3,)),
        pltpu.SemaphoreType.DMA((3,)),
    ],
)
def _sc_gather(a_hbm, b_hbm, src_hbm, dst_hbm, gs_hbm, gd_hbm,
               src_v, dst_v, bufs, bufd, gsem_s, gsem_d, wsem_s, wsem_d):
    wid = lax.axis_index("s") * 2 + lax.axis_index("c")
    pltpu.sync_copy(src_hbm.at[wid], src_v)
    pltpu.sync_copy(dst_hbm.at[wid], dst_v)

    def start_gather(j, p):
        pltpu.async_copy(a_hbm.at[src_v.at[j]], bufs.at[p], gsem_s.at[p])
        pltpu.async_copy(b_hbm.at[dst_v.at[j]], bufd.at[p], gsem_d.at[p])

    def wait_gather(p):
        pltpu.make_async_copy(a_hbm.at[src_v.at[0]], bufs.at[p],
                              gsem_s.at[p]).wait()
        pltpu.make_async_copy(b_hbm.at[dst_v.at[0]], bufd.at[p],
                              gsem_d.at[p]).wait()

    def start_write(j, p):
        base = wid * EPW + j * CH
        pltpu.async_copy(bufs.at[p], gs_hbm.at[pl.ds(base, CH)], wsem_s.at[p])
        pltpu.async_copy(bufd.at[p], gd_hbm.at[pl.ds(base, CH)], wsem_d.at[p])

    def wait_write(p):
        pltpu.make_async_copy(bufs.at[p], gs_hbm.at[pl.ds(0, CH)],
                              wsem_s.at[p]).wait()
        pltpu.make_async_copy(bufd.at[p], gd_hbm.at[pl.ds(0, CH)],
                              wsem_d.at[p]).wait()

    for jj in range(2):
        start_gather(jj, jj)

    def body(j, _):
        p = lax.rem(j, 3)
        q = lax.rem(j + 2, 3)
        wait_gather(p)

        @pl.when(j + 2 < NCH)
        def _():
            @pl.when(j >= 1)
            def _():
                wait_write(q)

            start_gather(j + 2, q)

        start_write(j, p)
        return 0

    lax.fori_loop(0, NCH, body, 0)
    for jj in range(3):
        wait_write(jj)


@functools.partial(
    pl.kernel,
    out_type=jax.ShapeDtypeStruct((2, N_NODES, D_H), f32),
    mesh=_MESH,
    scratch_types=[
        pltpu.VMEM((NCH, CH), jnp.int32),
        pltpu.VMEM((3, CH, D_H), f32),
        pltpu.VMEM_SHARED((N_NODES, D_H), f32),
        pltpu.SemaphoreType.DMA((3,)),
        pltpu.SemaphoreType.DMA((3,)),
    ],
)
def _sc_scatter(m_hbm, dst_hbm, zero_hbm, out_hbm, dst_v, buf, acc,
                rsem, ssem):
    cid = lax.axis_index("c")
    sid = lax.axis_index("s")
    wid = sid * 2 + cid

    @pl.when(sid == 0)
    def _():
        pltpu.sync_copy(zero_hbm, acc)

    pltpu.sync_copy(dst_hbm.at[wid], dst_v)
    plsc.subcore_barrier()

    def start_read(j, p):
        base = wid * EPW + j * CH
        pltpu.async_copy(m_hbm.at[pl.ds(base, CH)], buf.at[p], rsem.at[p])

    def wait_read(p):
        pltpu.make_async_copy(m_hbm.at[pl.ds(0, CH)], buf.at[p],
                              rsem.at[p]).wait()

    def wait_scatter(p):
        pltpu.make_async_copy(buf.at[p], acc.at[dst_v.at[0]],
                              ssem.at[p]).wait()

    for jj in range(2):
        start_read(jj, jj)

    def body(j, _):
        p = lax.rem(j, 3)
        q = lax.rem(j + 2, 3)
        wait_read(p)

        @pl.when(j + 2 < NCH)
        def _():
            @pl.when(j >= 1)
            def _():
                wait_scatter(q)

            start_read(j + 2, q)

        pltpu.async_copy(buf.at[p], acc.at[dst_v.at[j]], ssem.at[p],
                         add=True)
        return 0

    lax.fori_loop(0, NCH, body, 0)
    for jj in range(3):
        wait_scatter(jj)
    plsc.subcore_barrier()

    @pl.when(sid == 0)
    def _():
        pltpu.sync_copy(acc, out_hbm.at[cid])


def _ab(h, w1, b1):
    a = jnp.dot(h, w1[:D_H], preferred_element_type=f32) + b1
    b = jnp.dot(h, w1[D_H:], preferred_element_type=f32)
    return a, b


def _embed_body(x_ref, w_ref, b_ref, w1_ref, b1_ref, h_ref, a_ref, bb_ref):
    h = jnp.dot(x_ref[...], w_ref[...], preferred_element_type=f32) + b_ref[...]
    h_ref[...] = h
    a_ref[...], bb_ref[...] = _ab(h, w1_ref[...], b1_ref[...])


def _embed_pre(x, Wemb, bemb, W1, b1):
    return pl.pallas_call(
        _embed_body,
        out_shape=(
            jax.ShapeDtypeStruct((N_NODES, D_H), f32),
            jax.ShapeDtypeStruct((N_NODES, 2 * D_H), f32),
            jax.ShapeDtypeStruct((N_NODES, 2 * D_H), f32),
        ),
    )(x, Wemb, bemb.reshape(1, D_H), W1, b1.reshape(1, 2 * D_H))


BE = 3200


def _edge_body(gs_ref, gd_ref, w2_ref, b2_ref, w3_ref, b3_ref, o_ref):
    m1 = _gate(gs_ref[...] + gd_ref[...])
    m2 = _gate(jnp.dot(m1, w2_ref[...], preferred_element_type=f32)
               + b2_ref[...])
    o_ref[...] = (jnp.dot(m2, w3_ref[...], preferred_element_type=f32)
                  + b3_ref[...])


def _edge_mlp(gs, gd, W2, b2, W3, b3):
    nblk = N_EDGES // BE
    eb = pl.BlockSpec((BE, 2 * D_H), lambda i: (i, 0))
    full = lambda *shape: pl.BlockSpec(shape, lambda i: (0,) * len(shape))
    return pl.pallas_call(
        _edge_body,
        grid=(nblk,),
        in_specs=[eb, eb,
                  full(D_H, 2 * D_H), full(1, 2 * D_H),
                  full(D_H, D_H), full(1, D_H)],
        out_specs=pl.BlockSpec((BE, D_H), lambda i: (i, 0)),
        out_shape=jax.ShapeDtypeStruct((N_EDGES, D_H), f32),
    )(gs, gd, W2, b2.reshape(1, -1), W3, b3.reshape(1, -1))


def _node_body(s_ref, h_ref, wn_ref, bn_ref, w1_ref, b1_ref,
               o_ref, a_ref, bb_ref):
    scale = 1.0 / (N_EDGES - 1)
    m_i = (s_ref[0] + s_ref[1]) * scale
    h = h_ref[...]
    wn = wn_ref[...]
    h_new = (jnp.dot(m_i, wn[:D_H], preferred_element_type=f32)
             + jnp.dot(h, wn[D_H:], preferred_element_type=f32)
             + bn_ref[...] + h)
    o_ref[...] = h_new
    a_ref[...], bb_ref[...] = _ab(h_new, w1_ref[...], b1_ref[...])


def _node_update_pre(s_parts, h, Wn, bn, W1, b1):
    return pl.pallas_call(
        _node_body,
        out_shape=(
            jax.ShapeDtypeStruct((N_NODES, D_H), f32),
            jax.ShapeDtypeStruct((N_NODES, 2 * D_H), f32),
            jax.ShapeDtypeStruct((N_NODES, 2 * D_H), f32),
        ),
    )(s_parts, h, Wn, bn.reshape(1, D_H), W1, b1.reshape(1, 2 * D_H))


def _final_body(s_ref, h_ref, wn_ref, bn_ref, wd_ref, bd_ref,
                wdf_ref, bdf_ref, o_ref):
    scale = 1.0 / (N_EDGES - 1)
    m_i = (s_ref[0] + s_ref[1]) * scale
    h = h_ref[...]
    wn = wn_ref[...]
    h = (jnp.dot(m_i, wn[:D_H], preferred_element_type=f32)
         + jnp.dot(h, wn[D_H:], preferred_element_type=f32)
         + bn_ref[...] + h)
    for b in range(N_BLOCKS):
        h = _gate(jnp.dot(h, wd_ref[b], preferred_element_type=f32)
                  + bd_ref[b])
    o_ref[...] = (jnp.sum(h * wdf_ref[...], axis=1, keepdims=True)
                  + bdf_ref[...])


def _final_decode(s_parts, h, Wn, bn, Wd, bd, Wdf, bdf):
    return pl.pallas_call(
        _final_body,
        out_shape=jax.ShapeDtypeStruct((N_NODES, N_OUT), f32),
    )(s_parts, h, Wn, bn.reshape(1, D_H), Wd,
      bd.reshape(N_BLOCKS, 1, 2 * D_H), Wdf.reshape(1, D_H),
      bdf.reshape(1, N_OUT))


def kernel(x, edge_index, Wemb, bemb, We1, be1, We2, be2, We3, be3,
           Wn, bn, Wd, bd, Wdf, bdf):
    src = edge_index[0].astype(jnp.int32).reshape(NW, NCH, CH)
    dst = edge_index[1].astype(jnp.int32).reshape(NW, NCH, CH)
    zeros = jnp.zeros((N_NODES, D_H), dtype=f32)

    h, a, b = _embed_pre(x, Wemb, bemb, We1[0], be1[0])
    for s in range(N_STEPS):
        gs, gd = _sc_gather(a, b, src, dst)
        m3 = _edge_mlp(gs, gd, We2[s], be2[s], We3[s], be3[s])
        s_parts = _sc_scatter(m3, dst, zeros)
        if s + 1 < N_STEPS:
            h, a, b = _node_update_pre(s_parts, h, Wn[s], bn[s],
                                       We1[s + 1], be1[s + 1])
        else:
            return _final_decode(s_parts, h, Wn[s], bn[s], Wd, bd, Wdf, bdf)


# speedup vs baseline: 3.9249x; 1.0110x over previous
import functools

import jax
import jax.numpy as jnp
from jax import lax
from jax.experimental import pallas as pl
from jax.experimental.pallas import tpu as pltpu
from jax.experimental.pallas import tpu_sc as plsc

N_NODES = 10000
N_EDGES = 320000
D_IN = 128
D_H = 64
N_STEPS = 3
N_BLOCKS = 3
N_OUT = 1

NW = 32
EPW = N_EDGES // NW
CH = 80
NCH = EPW // CH

_MESH = plsc.VectorSubcoreMesh(core_axis_name="c", subcore_axis_name="s",
                               num_cores=2, num_subcores=16)

f32 = jnp.float32


def _gate(z):
    v, g = jnp.split(z, 2, axis=-1)
    return (0.25 * v) * (1.0 + jnp.tanh(0.5 * v)) * (1.0 + jnp.tanh(0.5 * g))


@functools.partial(
    pl.kernel,
    out_type=(
        jax.ShapeDtypeStruct((N_EDGES, 2 * D_H), f32),
        jax.ShapeDtypeStruct((N_EDGES, 2 * D_H), f32),
    ),
    mesh=_MESH,
    scratch_types=[
        pltpu.VMEM((NCH, CH), jnp.int32),
        pltpu.VMEM((NCH, CH), jnp.int32),
        pltpu.VMEM((3, CH, 2 * D_H), f32),
        pltpu.VMEM((3, CH, 2 * D_H), f32),
        pltpu.SemaphoreType.DMA((3,)),
        pltpu.SemaphoreType.DMA((3,)),
        pltpu.SemaphoreType.DMA((3,)),
        pltpu.SemaphoreType.DMA((3,)),
    ],
)
def _sc_gather(a_hbm, b_hbm, src_hbm, dst_hbm, gs_hbm, gd_hbm,
               src_v, dst_v, bufs, bufd, gsem_s, gsem_d, wsem_s, wsem_d):
    wid = lax.axis_index("s") * 2 + lax.axis_index("c")
    pltpu.sync_copy(src_hbm.at[wid], src_v)
    pltpu.sync_copy(dst_hbm.at[wid], dst_v)

    def start_gather(j, p):
        pltpu.async_copy(a_hbm.at[src_v.at[j]], bufs.at[p], gsem_s.at[p])
        pltpu.async_copy(b_hbm.at[dst_v.at[j]], bufd.at[p], gsem_d.at[p])

    def wait_gather(p):
        pltpu.make_async_copy(a_hbm.at[src_v.at[0]], bufs.at[p],
                              gsem_s.at[p]).wait()
        pltpu.make_async_copy(b_hbm.at[dst_v.at[0]], bufd.at[p],
                              gsem_d.at[p]).wait()

    def start_write(j, p):
        base = wid * EPW + j * CH
        pltpu.async_copy(bufs.at[p], gs_hbm.at[pl.ds(base, CH)], wsem_s.at[p])
        pltpu.async_copy(bufd.at[p], gd_hbm.at[pl.ds(base, CH)], wsem_d.at[p])

    def wait_write(p):
        pltpu.make_async_copy(bufs.at[p], gs_hbm.at[pl.ds(0, CH)],
                              wsem_s.at[p]).wait()
        pltpu.make_async_copy(bufd.at[p], gd_hbm.at[pl.ds(0, CH)],
                              wsem_d.at[p]).wait()

    for jj in range(2):
        start_gather(jj, jj)

    def body(j, _):
        p = lax.rem(j, 3)
        q = lax.rem(j + 2, 3)
        wait_gather(p)

        @pl.when(j + 2 < NCH)
        def _():
            @pl.when(j >= 1)
            def _():
                wait_write(q)

            start_gather(j + 2, q)

        start_write(j, p)
        return 0

    lax.fori_loop(0, NCH, body, 0)
    for jj in range(3):
        wait_write(jj)


@functools.partial(
    pl.kernel,
    out_type=jax.ShapeDtypeStruct((2, N_NODES, D_H), f32),
    mesh=_MESH,
    scratch_types=[
        pltpu.VMEM((NCH, CH), jnp.int32),
        pltpu.VMEM((3, CH, D_H), f32),
        pltpu.VMEM_SHARED((N_NODES, D_H), f32),
        pltpu.SemaphoreType.DMA((3,)),
        pltpu.SemaphoreType.DMA((3,)),
    ],
)
def _sc_scatter(m_hbm, dst_hbm, zero_hbm, out_hbm, dst_v, buf, acc,
                rsem, ssem):
    cid = lax.axis_index("c")
    sid = lax.axis_index("s")
    wid = sid * 2 + cid

    @pl.when(sid == 0)
    def _():
        pltpu.sync_copy(zero_hbm, acc)

    pltpu.sync_copy(dst_hbm.at[wid], dst_v)
    plsc.subcore_barrier()

    def start_read(j, p):
        base = wid * EPW + j * CH
        pltpu.async_copy(m_hbm.at[pl.ds(base, CH)], buf.at[p], rsem.at[p])

    def wait_read(p):
        pltpu.make_async_copy(m_hbm.at[pl.ds(0, CH)], buf.at[p],
                              rsem.at[p]).wait()

    def wait_scatter(p):
        pltpu.make_async_copy(buf.at[p], acc.at[dst_v.at[0]],
                              ssem.at[p]).wait()

    for jj in range(2):
        start_read(jj, jj)

    def body(j, _):
        p = lax.rem(j, 3)
        q = lax.rem(j + 2, 3)
        wait_read(p)

        @pl.when(j + 2 < NCH)
        def _():
            @pl.when(j >= 1)
            def _():
                wait_scatter(q)

            start_read(j + 2, q)

        pltpu.async_copy(buf.at[p], acc.at[dst_v.at[j]], ssem.at[p],
                         add=True)
        return 0

    lax.fori_loop(0, NCH, body, 0)
    for jj in range(3):
        wait_scatter(jj)
    plsc.subcore_barrier()

    @pl.when(sid == 0)
    def _():
        pltpu.sync_copy(acc, out_hbm.at[cid])


def _ab(h, w1, b1):
    a = jnp.dot(h, w1[:D_H], preferred_element_type=f32) + b1
    b = jnp.dot(h, w1[D_H:], preferred_element_type=f32)
    return a, b


def _embed_body(x_ref, w_ref, b_ref, w1_ref, b1_ref, h_ref, a_ref, bb_ref):
    h = jnp.dot(x_ref[...], w_ref[...], preferred_element_type=f32) + b_ref[...]
    h_ref[...] = h
    a_ref[...], bb_ref[...] = _ab(h, w1_ref[...], b1_ref[...])


def _embed_pre(x, Wemb, bemb, W1, b1):
    return pl.pallas_call(
        _embed_body,
        out_shape=(
            jax.ShapeDtypeStruct((N_NODES, D_H), f32),
            jax.ShapeDtypeStruct((N_NODES, 2 * D_H), f32),
            jax.ShapeDtypeStruct((N_NODES, 2 * D_H), f32),
        ),
    )(x, Wemb, bemb.reshape(1, D_H), W1, b1.reshape(1, 2 * D_H))


BE = 3200


def _edge_body(gs_ref, gd_ref, w2_ref, b2_ref, w3_ref, b3_ref, o_ref):
    m1 = _gate(gs_ref[...] + gd_ref[...])
    m2 = _gate(jnp.dot(m1, w2_ref[...], preferred_element_type=f32)
               + b2_ref[...])
    o_ref[...] = (jnp.dot(m2, w3_ref[...], preferred_element_type=f32)
                  + b3_ref[...])


def _edge_mlp(gs, gd, W2, b2, W3, b3):
    nblk = N_EDGES // BE
    eb = pl.BlockSpec((BE, 2 * D_H), lambda i: (i, 0))
    full = lambda *shape: pl.BlockSpec(shape, lambda i: (0,) * len(shape))
    return pl.pallas_call(
        _edge_body,
        grid=(nblk,),
        in_specs=[eb, eb,
                  full(D_H, 2 * D_H), full(1, 2 * D_H),
                  full(D_H, D_H), full(1, D_H)],
        out_specs=pl.BlockSpec((BE, D_H), lambda i: (i, 0)),
        out_shape=jax.ShapeDtypeStruct((N_EDGES, D_H), f32),
    )(gs, gd, W2, b2.reshape(1, -1), W3, b3.reshape(1, -1))


def _node_body(s_ref, h_ref, wn_ref, bn_ref, w1_ref, b1_ref,
               o_ref, a_ref, bb_ref):
    scale = 1.0 / (N_EDGES - 1)
    m_i = (s_ref[0] + s_ref[1]) * scale
    h = h_ref[...]
    wn = wn_ref[...]
    h_new = (jnp.dot(m_i, wn[:D_H], preferred_element_type=f32)
             + jnp.dot(h, wn[D_H:], preferred_element_type=f32)
             + bn_ref[...] + h)
    o_ref[...] = h_new
    a_ref[...], bb_ref[...] = _ab(h_new, w1_ref[...], b1_ref[...])


def _node_update_pre(s_parts, h, Wn, bn, W1, b1):
    return pl.pallas_call(
        _node_body,
        out_shape=(
            jax.ShapeDtypeStruct((N_NODES, D_H), f32),
            jax.ShapeDtypeStruct((N_NODES, 2 * D_H), f32),
            jax.ShapeDtypeStruct((N_NODES, 2 * D_H), f32),
        ),
    )(s_parts, h, Wn, bn.reshape(1, D_H), W1, b1.reshape(1, 2 * D_H))


def _final_body(s_ref, h_ref, wn_ref, bn_ref, wd_ref, bd_ref,
                wdf_ref, bdf_ref, o_ref):
    scale = 1.0 / (N_EDGES - 1)
    m_i = (s_ref[0] + s_ref[1]) * scale
    h = h_ref[...]
    wn = wn_ref[...]
    h = (jnp.dot(m_i, wn[:D_H], preferred_element_type=f32)
         + jnp.dot(h, wn[D_H:], preferred_element_type=f32)
         + bn_ref[...] + h)
    for b in range(N_BLOCKS):
        h = _gate(jnp.dot(h, wd_ref[b], preferred_element_type=f32)
                  + bd_ref[b])
    o_ref[...] = (jnp.sum(h * wdf_ref[...], axis=1, keepdims=True)
                  + bdf_ref[...])


def _final_decode(s_parts, h, Wn, bn, Wd, bd, Wdf, bdf):
    return pl.pallas_call(
        _final_body,
        out_shape=jax.ShapeDtypeStruct((N_NODES, N_OUT), f32),
    )(s_parts, h, Wn, bn.reshape(1, D_H), Wd,
      bd.reshape(N_BLOCKS, 1, 2 * D_H), Wdf.reshape(1, D_H),
      bdf.reshape(1, N_OUT))


def kernel(x, edge_index, Wemb, bemb, We1, be1, We2, be2, We3, be3,
           Wn, bn, Wd, bd, Wdf, bdf):
    src = edge_index[0].astype(jnp.int32).reshape(NW, NCH, CH)
    dst = edge_index[1].astype(jnp.int32).reshape(NW, NCH, CH)
    zeros = jnp.zeros((N_NODES, D_H), dtype=f32)

    h, a, b = _embed_pre(x, Wemb, bemb, We1[0], be1[0])
    for s in range(N_STEPS):
        gs, gd = _sc_gather(a, b, src, dst)
        m3 = _edge_mlp(gs, gd, We2[s], be2[s], We3[s], be3[s])
        s_parts = _sc_scatter(m3, dst, zeros)
        if s + 1 < N_STEPS:
            h, a, b = _node_update_pre(s_parts, h, Wn[s], bn[s],
                                       We1[s + 1], be1[s + 1])
        else:
            return _final_decode(s_parts, h, Wn[s], bn[s], Wd, bd, Wdf, bdf)


# speedup vs baseline: 4.1461x; 1.0564x over previous
import functools

import jax
import jax.numpy as jnp
from jax import lax
from jax.experimental import pallas as pl
from jax.experimental.pallas import tpu as pltpu
from jax.experimental.pallas import tpu_sc as plsc

N_NODES = 10000
N_EDGES = 320000
D_IN = 128
D_H = 64
N_STEPS = 3
N_BLOCKS = 3
N_OUT = 1

NW = 32
EPW = N_EDGES // NW
CH = 80
NCH = EPW // CH

_MESH = plsc.VectorSubcoreMesh(core_axis_name="c", subcore_axis_name="s",
                               num_cores=2, num_subcores=16)

f32 = jnp.float32


def _gate(z):
    v, g = jnp.split(z, 2, axis=-1)
    return (0.25 * v) * (1.0 + jnp.tanh(0.5 * v)) * (1.0 + jnp.tanh(0.5 * g))


@functools.partial(
    pl.kernel,
    out_type=(
        jax.ShapeDtypeStruct((N_EDGES, 2 * D_H), f32),
        jax.ShapeDtypeStruct((N_EDGES, 2 * D_H), f32),
    ),
    mesh=_MESH,
    scratch_types=[
        pltpu.VMEM((NCH, CH), jnp.int32),
        pltpu.VMEM((NCH, CH), jnp.int32),
        pltpu.VMEM((3, CH, 2 * D_H), f32),
        pltpu.VMEM((3, CH, 2 * D_H), f32),
        pltpu.SemaphoreType.DMA((3,)),
        pltpu.SemaphoreType.DMA((3,)),
        pltpu.SemaphoreType.DMA((3,)),
        pltpu.SemaphoreType.DMA((3,)),
    ],
)
def _sc_gather(a_hbm, b_hbm, src_hbm, dst_hbm, gs_hbm, gd_hbm,
               src_v, dst_v, bufs, bufd, gsem_s, gsem_d, wsem_s, wsem_d):
    wid = lax.axis_index("s") * 2 + lax.axis_index("c")
    pltpu.sync_copy(src_hbm.at[wid], src_v)
    pltpu.sync_copy(dst_hbm.at[wid], dst_v)

    def start_gather(j, p):
        pltpu.async_copy(a_hbm.at[src_v.at[j]], bufs.at[p], gsem_s.at[p])
        pltpu.async_copy(b_hbm.at[dst_v.at[j]], bufd.at[p], gsem_d.at[p])

    def wait_gather(p):
        pltpu.make_async_copy(a_hbm.at[src_v.at[0]], bufs.at[p],
                              gsem_s.at[p]).wait()
        pltpu.make_async_copy(b_hbm.at[dst_v.at[0]], bufd.at[p],
                              gsem_d.at[p]).wait()

    def start_write(j, p):
        base = wid * EPW + j * CH
        pltpu.async_copy(bufs.at[p], gs_hbm.at[pl.ds(base, CH)], wsem_s.at[p])
        pltpu.async_copy(bufd.at[p], gd_hbm.at[pl.ds(base, CH)], wsem_d.at[p])

    def wait_write(p):
        pltpu.make_async_copy(bufs.at[p], gs_hbm.at[pl.ds(0, CH)],
                              wsem_s.at[p]).wait()
        pltpu.make_async_copy(bufd.at[p], gd_hbm.at[pl.ds(0, CH)],
                              wsem_d.at[p]).wait()

    for jj in range(2):
        start_gather(jj, jj)

    def body(j, _):
        p = lax.rem(j, 3)
        q = lax.rem(j + 2, 3)
        wait_gather(p)

        @pl.when(j + 2 < NCH)
        def _():
            @pl.when(j >= 1)
            def _():
                wait_write(q)

            start_gather(j + 2, q)

        start_write(j, p)
        return 0

    lax.fori_loop(0, NCH, body, 0)
    for jj in range(3):
        wait_write(jj)


@functools.partial(
    pl.kernel,
    out_type=jax.ShapeDtypeStruct((2, N_NODES, D_H), f32),
    mesh=_MESH,
    scratch_types=[
        pltpu.VMEM((NCH, CH), jnp.int32),
        pltpu.VMEM((3, CH, D_H), f32),
        pltpu.VMEM_SHARED((N_NODES, D_H), f32),
        pltpu.SemaphoreType.DMA((3,)),
        pltpu.SemaphoreType.DMA((3,)),
    ],
)
def _sc_scatter(m_hbm, dst_hbm, zero_hbm, out_hbm, dst_v, buf, acc,
                rsem, ssem):
    cid = lax.axis_index("c")
    sid = lax.axis_index("s")
    wid = sid * 2 + cid

    @pl.when(sid == 0)
    def _():
        pltpu.sync_copy(zero_hbm, acc)

    pltpu.sync_copy(dst_hbm.at[wid], dst_v)
    plsc.subcore_barrier()

    def start_read(j, p):
        base = wid * EPW + j * CH
        pltpu.async_copy(m_hbm.at[pl.ds(base, CH)], buf.at[p], rsem.at[p])

    def wait_read(p):
        pltpu.make_async_copy(m_hbm.at[pl.ds(0, CH)], buf.at[p],
                              rsem.at[p]).wait()

    def wait_scatter(p):
        pltpu.make_async_copy(buf.at[p], acc.at[dst_v.at[0]],
                              ssem.at[p]).wait()

    for jj in range(2):
        start_read(jj, jj)

    def body(j, _):
        p = lax.rem(j, 3)
        q = lax.rem(j + 2, 3)
        wait_read(p)

        @pl.when(j + 2 < NCH)
        def _():
            @pl.when(j >= 1)
            def _():
                wait_scatter(q)

            start_read(j + 2, q)

        pltpu.async_copy(buf.at[p], acc.at[dst_v.at[j]], ssem.at[p],
                         add=True)
        return 0

    lax.fori_loop(0, NCH, body, 0)
    for jj in range(3):
        wait_scatter(jj)
    plsc.subcore_barrier()

    @pl.when(sid == 0)
    def _():
        pltpu.sync_copy(acc, out_hbm.at[cid])


def _ab(h, w1, b1):
    a = jnp.dot(h, w1[:D_H], preferred_element_type=f32) + b1
    b = jnp.dot(h, w1[D_H:], preferred_element_type=f32)
    return a, b


def _embed_body(x_ref, w_ref, b_ref, w1_ref, b1_ref, h_ref, a_ref, bb_ref):
    h = jnp.dot(x_ref[...], w_ref[...], preferred_element_type=f32) + b_ref[...]
    h_ref[...] = h
    a_ref[...], bb_ref[...] = _ab(h, w1_ref[...], b1_ref[...])


def _embed_pre(x, Wemb, bemb, W1, b1):
    return pl.pallas_call(
        _embed_body,
        out_shape=(
            jax.ShapeDtypeStruct((N_NODES, D_H), f32),
            jax.ShapeDtypeStruct((N_NODES, 2 * D_H), f32),
            jax.ShapeDtypeStruct((N_NODES, 2 * D_H), f32),
        ),
    )(x, Wemb, bemb.reshape(1, D_H), W1, b1.reshape(1, 2 * D_H))


BE = 6400


def _edge_body(gs_ref, gd_ref, w2_ref, b2_ref, w3_ref, b3_ref, o_ref):
    m1 = _gate(gs_ref[...] + gd_ref[...])
    m2 = _gate(jnp.dot(m1, w2_ref[...], preferred_element_type=f32)
               + b2_ref[...])
    o_ref[...] = (jnp.dot(m2, w3_ref[...], preferred_element_type=f32)
                  + b3_ref[...])


def _edge_mlp(gs, gd, W2, b2, W3, b3):
    nblk = N_EDGES // BE
    eb = pl.BlockSpec((BE, 2 * D_H), lambda i: (i, 0))
    full = lambda *shape: pl.BlockSpec(shape, lambda i: (0,) * len(shape))
    return pl.pallas_call(
        _edge_body,
        grid=(nblk,),
        in_specs=[eb, eb,
                  full(D_H, 2 * D_H), full(1, 2 * D_H),
                  full(D_H, D_H), full(1, D_H)],
        out_specs=pl.BlockSpec((BE, D_H), lambda i: (i, 0)),
        out_shape=jax.ShapeDtypeStruct((N_EDGES, D_H), f32),
    )(gs, gd, W2, b2.reshape(1, -1), W3, b3.reshape(1, -1))


def _node_body(s_ref, h_ref, wn_ref, bn_ref, w1_ref, b1_ref,
               o_ref, a_ref, bb_ref):
    scale = 1.0 / (N_EDGES - 1)
    m_i = (s_ref[0] + s_ref[1]) * scale
    h = h_ref[...]
    wn = wn_ref[...]
    h_new = (jnp.dot(m_i, wn[:D_H], preferred_element_type=f32)
             + jnp.dot(h, wn[D_H:], preferred_element_type=f32)
             + bn_ref[...] + h)
    o_ref[...] = h_new
    a_ref[...], bb_ref[...] = _ab(h_new, w1_ref[...], b1_ref[...])


def _node_update_pre(s_parts, h, Wn, bn, W1, b1):
    return pl.pallas_call(
        _node_body,
        out_shape=(
            jax.ShapeDtypeStruct((N_NODES, D_H), f32),
            jax.ShapeDtypeStruct((N_NODES, 2 * D_H), f32),
            jax.ShapeDtypeStruct((N_NODES, 2 * D_H), f32),
        ),
    )(s_parts, h, Wn, bn.reshape(1, D_H), W1, b1.reshape(1, 2 * D_H))


def _final_body(s_ref, h_ref, wn_ref, bn_ref, wd_ref, bd_ref,
                wdf_ref, bdf_ref, o_ref):
    scale = 1.0 / (N_EDGES - 1)
    m_i = (s_ref[0] + s_ref[1]) * scale
    h = h_ref[...]
    wn = wn_ref[...]
    h = (jnp.dot(m_i, wn[:D_H], preferred_element_type=f32)
         + jnp.dot(h, wn[D_H:], preferred_element_type=f32)
         + bn_ref[...] + h)
    for b in range(N_BLOCKS):
        h = _gate(jnp.dot(h, wd_ref[b], preferred_element_type=f32)
                  + bd_ref[b])
    o_ref[...] = (jnp.sum(h * wdf_ref[...], axis=1, keepdims=True)
                  + bdf_ref[...])


def _final_decode(s_parts, h, Wn, bn, Wd, bd, Wdf, bdf):
    return pl.pallas_call(
        _final_body,
        out_shape=jax.ShapeDtypeStruct((N_NODES, N_OUT), f32),
    )(s_parts, h, Wn, bn.reshape(1, D_H), Wd,
      bd.reshape(N_BLOCKS, 1, 2 * D_H), Wdf.reshape(1, D_H),
      bdf.reshape(1, N_OUT))


def kernel(x, edge_index, Wemb, bemb, We1, be1, We2, be2, We3, be3,
           Wn, bn, Wd, bd, Wdf, bdf):
    src = edge_index[0].astype(jnp.int32).reshape(NW, NCH, CH)
    dst = edge_index[1].astype(jnp.int32).reshape(NW, NCH, CH)
    zeros = jnp.zeros((N_NODES, D_H), dtype=f32)

    h, a, b = _embed_pre(x, Wemb, bemb, We1[0], be1[0])
    for s in range(N_STEPS):
        gs, gd = _sc_gather(a, b, src, dst)
        m3 = _edge_mlp(gs, gd, We2[s], be2[s], We3[s], be3[s])
        s_parts = _sc_scatter(m3, dst, zeros)
        if s + 1 < N_STEPS:
            h, a, b = _node_update_pre(s_parts, h, Wn[s], bn[s],
                                       We1[s + 1], be1[s + 1])
        else:
            return _final_decode(s_parts, h, Wn[s], bn[s], Wd, bd, Wdf, bdf)


# speedup vs baseline: 4.1559x; 1.0024x over previous
import functools

import jax
import jax.numpy as jnp
from jax import lax
from jax.experimental import pallas as pl
from jax.experimental.pallas import tpu as pltpu
from jax.experimental.pallas import tpu_sc as plsc

N_NODES = 10000
N_EDGES = 320000
D_IN = 128
D_H = 64
N_STEPS = 3
N_BLOCKS = 3
N_OUT = 1

NW = 32
EPW = N_EDGES // NW
CH = 80
NCH = EPW // CH

_MESH = plsc.VectorSubcoreMesh(core_axis_name="c", subcore_axis_name="s",
                               num_cores=2, num_subcores=16)

f32 = jnp.float32


def _gate(z):
    v, g = jnp.split(z, 2, axis=-1)
    return (0.25 * v) * (1.0 + jnp.tanh(0.5 * v)) * (1.0 + jnp.tanh(0.5 * g))


@functools.partial(
    pl.kernel,
    out_type=(
        jax.ShapeDtypeStruct((N_EDGES, 2 * D_H), f32),
        jax.ShapeDtypeStruct((N_EDGES, 2 * D_H), f32),
    ),
    mesh=_MESH,
    scratch_types=[
        pltpu.VMEM((NCH, CH), jnp.int32),
        pltpu.VMEM((NCH, CH), jnp.int32),
        pltpu.VMEM((3, CH, 2 * D_H), f32),
        pltpu.VMEM((3, CH, 2 * D_H), f32),
        pltpu.SemaphoreType.DMA((3,)),
        pltpu.SemaphoreType.DMA((3,)),
        pltpu.SemaphoreType.DMA((3,)),
        pltpu.SemaphoreType.DMA((3,)),
    ],
)
def _sc_gather(a_hbm, b_hbm, src_hbm, dst_hbm, gs_hbm, gd_hbm,
               src_v, dst_v, bufs, bufd, gsem_s, gsem_d, wsem_s, wsem_d):
    wid = lax.axis_index("s") * 2 + lax.axis_index("c")
    pltpu.sync_copy(src_hbm.at[wid], src_v)
    pltpu.sync_copy(dst_hbm.at[wid], dst_v)

    def start_gather(j, p):
        pltpu.async_copy(a_hbm.at[src_v.at[j]], bufs.at[p], gsem_s.at[p])
        pltpu.async_copy(b_hbm.at[dst_v.at[j]], bufd.at[p], gsem_d.at[p])

    def wait_gather(p):
        pltpu.make_async_copy(a_hbm.at[src_v.at[0]], bufs.at[p],
                              gsem_s.at[p]).wait()
        pltpu.make_async_copy(b_hbm.at[dst_v.at[0]], bufd.at[p],
                              gsem_d.at[p]).wait()

    def start_write(j, p):
        base = wid * EPW + j * CH
        pltpu.async_copy(bufs.at[p], gs_hbm.at[pl.ds(base, CH)], wsem_s.at[p])
        pltpu.async_copy(bufd.at[p], gd_hbm.at[pl.ds(base, CH)], wsem_d.at[p])

    def wait_write(p):
        pltpu.make_async_copy(bufs.at[p], gs_hbm.at[pl.ds(0, CH)],
                              wsem_s.at[p]).wait()
        pltpu.make_async_copy(bufd.at[p], gd_hbm.at[pl.ds(0, CH)],
                              wsem_d.at[p]).wait()

    for jj in range(2):
        start_gather(jj, jj)

    def body(j, _):
        p = lax.rem(j, 3)
        q = lax.rem(j + 2, 3)
        wait_gather(p)

        @pl.when(j + 2 < NCH)
        def _():
            @pl.when(j >= 1)
            def _():
                wait_write(q)

            start_gather(j + 2, q)

        start_write(j, p)
        return 0

    lax.fori_loop(0, NCH, body, 0)
    for jj in range(3):
        wait_write(jj)


@functools.partial(
    pl.kernel,
    out_type=jax.ShapeDtypeStruct((2, N_NODES, D_H), f32),
    mesh=_MESH,
    scratch_types=[
        pltpu.VMEM((NCH, CH), jnp.int32),
        pltpu.VMEM((3, CH, D_H), f32),
        pltpu.VMEM_SHARED((N_NODES, D_H), f32),
        pltpu.SemaphoreType.DMA((3,)),
        pltpu.SemaphoreType.DMA((3,)),
    ],
)
def _sc_scatter(m_hbm, dst_hbm, zero_hbm, out_hbm, dst_v, buf, acc,
                rsem, ssem):
    cid = lax.axis_index("c")
    sid = lax.axis_index("s")
    wid = sid * 2 + cid

    @pl.when(sid == 0)
    def _():
        pltpu.sync_copy(zero_hbm, acc)

    pltpu.sync_copy(dst_hbm.at[wid], dst_v)
    plsc.subcore_barrier()

    def start_read(j, p):
        base = wid * EPW + j * CH
        pltpu.async_copy(m_hbm.at[pl.ds(base, CH)], buf.at[p], rsem.at[p])

    def wait_read(p):
        pltpu.make_async_copy(m_hbm.at[pl.ds(0, CH)], buf.at[p],
                              rsem.at[p]).wait()

    def wait_scatter(p):
        pltpu.make_async_copy(buf.at[p], acc.at[dst_v.at[0]],
                              ssem.at[p]).wait()

    for jj in range(2):
        start_read(jj, jj)

    def body(j, _):
        p = lax.rem(j, 3)
        q = lax.rem(j + 2, 3)
        wait_read(p)

        @pl.when(j + 2 < NCH)
        def _():
            @pl.when(j >= 1)
            def _():
                wait_scatter(q)

            start_read(j + 2, q)

        pltpu.async_copy(buf.at[p], acc.at[dst_v.at[j]], ssem.at[p],
                         add=True)
        return 0

    lax.fori_loop(0, NCH, body, 0)
    for jj in range(3):
        wait_scatter(jj)
    plsc.subcore_barrier()

    @pl.when(sid == 0)
    def _():
        pltpu.sync_copy(acc, out_hbm.at[cid])


def _ab(h, w1, b1):
    a = jnp.dot(h, w1[:D_H], preferred_element_type=f32) + b1
    b = jnp.dot(h, w1[D_H:], preferred_element_type=f32)
    return a, b


def _embed_body(x_ref, w_ref, b_ref, w1_ref, b1_ref, h_ref, a_ref, bb_ref):
    h = jnp.dot(x_ref[...], w_ref[...], preferred_element_type=f32) + b_ref[...]
    h_ref[...] = h
    a_ref[...], bb_ref[...] = _ab(h, w1_ref[...], b1_ref[...])


def _embed_pre(x, Wemb, bemb, W1, b1):
    return pl.pallas_call(
        _embed_body,
        out_shape=(
            jax.ShapeDtypeStruct((N_NODES, D_H), f32),
            jax.ShapeDtypeStruct((N_NODES, 2 * D_H), f32),
            jax.ShapeDtypeStruct((N_NODES, 2 * D_H), f32),
        ),
    )(x, Wemb, bemb.reshape(1, D_H), W1, b1.reshape(1, 2 * D_H))


BE = 8000


def _edge_body(gs_ref, gd_ref, w2_ref, b2_ref, w3_ref, b3_ref, o_ref):
    m1 = _gate(gs_ref[...] + gd_ref[...])
    m2 = _gate(jnp.dot(m1, w2_ref[...], preferred_element_type=f32)
               + b2_ref[...])
    o_ref[...] = (jnp.dot(m2, w3_ref[...], preferred_element_type=f32)
                  + b3_ref[...])


def _edge_mlp(gs, gd, W2, b2, W3, b3):
    nblk = N_EDGES // BE
    eb = pl.BlockSpec((BE, 2 * D_H), lambda i: (i, 0))
    full = lambda *shape: pl.BlockSpec(shape, lambda i: (0,) * len(shape))
    return pl.pallas_call(
        _edge_body,
        grid=(nblk,),
        in_specs=[eb, eb,
                  full(D_H, 2 * D_H), full(1, 2 * D_H),
                  full(D_H, D_H), full(1, D_H)],
        out_specs=pl.BlockSpec((BE, D_H), lambda i: (i, 0)),
        out_shape=jax.ShapeDtypeStruct((N_EDGES, D_H), f32),
    )(gs, gd, W2, b2.reshape(1, -1), W3, b3.reshape(1, -1))


def _node_body(s_ref, h_ref, wn_ref, bn_ref, w1_ref, b1_ref,
               o_ref, a_ref, bb_ref):
    scale = 1.0 / (N_EDGES - 1)
    m_i = (s_ref[0] + s_ref[1]) * scale
    h = h_ref[...]
    wn = wn_ref[...]
    h_new = (jnp.dot(m_i, wn[:D_H], preferred_element_type=f32)
             + jnp.dot(h, wn[D_H:], preferred_element_type=f32)
             + bn_ref[...] + h)
    o_ref[...] = h_new
    a_ref[...], bb_ref[...] = _ab(h_new, w1_ref[...], b1_ref[...])


def _node_update_pre(s_parts, h, Wn, bn, W1, b1):
    return pl.pallas_call(
        _node_body,
        out_shape=(
            jax.ShapeDtypeStruct((N_NODES, D_H), f32),
            jax.ShapeDtypeStruct((N_NODES, 2 * D_H), f32),
            jax.ShapeDtypeStruct((N_NODES, 2 * D_H), f32),
        ),
    )(s_parts, h, Wn, bn.reshape(1, D_H), W1, b1.reshape(1, 2 * D_H))


def _final_body(s_ref, h_ref, wn_ref, bn_ref, wd_ref, bd_ref,
                wdf_ref, bdf_ref, o_ref):
    scale = 1.0 / (N_EDGES - 1)
    m_i = (s_ref[0] + s_ref[1]) * scale
    h = h_ref[...]
    wn = wn_ref[...]
    h = (jnp.dot(m_i, wn[:D_H], preferred_element_type=f32)
         + jnp.dot(h, wn[D_H:], preferred_element_type=f32)
         + bn_ref[...] + h)
    for b in range(N_BLOCKS):
        h = _gate(jnp.dot(h, wd_ref[b], preferred_element_type=f32)
                  + bd_ref[b])
    o_ref[...] = (jnp.sum(h * wdf_ref[...], axis=1, keepdims=True)
                  + bdf_ref[...])


def _final_decode(s_parts, h, Wn, bn, Wd, bd, Wdf, bdf):
    return pl.pallas_call(
        _final_body,
        out_shape=jax.ShapeDtypeStruct((N_NODES, N_OUT), f32),
    )(s_parts, h, Wn, bn.reshape(1, D_H), Wd,
      bd.reshape(N_BLOCKS, 1, 2 * D_H), Wdf.reshape(1, D_H),
      bdf.reshape(1, N_OUT))


def kernel(x, edge_index, Wemb, bemb, We1, be1, We2, be2, We3, be3,
           Wn, bn, Wd, bd, Wdf, bdf):
    src = edge_index[0].astype(jnp.int32).reshape(NW, NCH, CH)
    dst = edge_index[1].astype(jnp.int32).reshape(NW, NCH, CH)
    zeros = jnp.zeros((N_NODES, D_H), dtype=f32)

    h, a, b = _embed_pre(x, Wemb, bemb, We1[0], be1[0])
    for s in range(N_STEPS):
        gs, gd = _sc_gather(a, b, src, dst)
        m3 = _edge_mlp(gs, gd, We2[s], be2[s], We3[s], be3[s])
        s_parts = _sc_scatter(m3, dst, zeros)
        if s + 1 < N_STEPS:
            h, a, b = _node_update_pre(s_parts, h, Wn[s], bn[s],
                                       We1[s + 1], be1[s + 1])
        else:
            return _final_decode(s_parts, h, Wn[s], bn[s], Wd, bd, Wdf, bdf)
